# Initial kernel scaffold; baseline (speedup 1.0000x reference)
import jax
import jax.numpy as jnp
from jax.experimental import pallas as pl


def kernel(x, pos, batch, params):
    raise NotImplementedError("write your pallas kernel here")


# speedup vs baseline: 4.2886x; 4.2886x over previous
import functools

import jax
import jax.numpy as jnp
import numpy as np
from jax import lax
from jax.experimental import pallas as pl
from jax.experimental.pallas import tpu as pltpu
from jax.experimental.pallas import tpu_sc as plsc

_NEG = 0.2
_F32 = jnp.float32


def _leaky(v):
    return jnp.where(v > 0, v, _NEG * v)


def _dot(a, b):
    return jnp.dot(a, b, preferred_element_type=_F32)


def _knn16(pos8, posT):
    N = pos8.shape[0]
    Bq = 256

    def body(q_ref, pt_ref, o_ref):
        pt = pt_ref[...]
        p2 = jnp.sum(pt * pt, axis=0, keepdims=True)
        q = q_ref[...]
        q2 = jnp.sum(q * q, axis=1, keepdims=True)
        d = q2 - 2.0 * _dot(q, pt) + p2
        iota = lax.broadcasted_iota(jnp.int32, (Bq, N), 1)
        inf = jnp.float32(np.inf)
        for t in range(16):
            vmin = jnp.min(d, axis=1, keepdims=True)
            am = jnp.min(jnp.where(d == vmin, iota, N), axis=1)
            o_ref[t, :] = am
            d = jnp.where(iota == am[:, None], inf, d)

    grid = (pl.cdiv(N, Bq),)
    return pl.pallas_call(
        body,
        grid=grid,
        in_specs=[
            pl.BlockSpec((Bq, 8), lambda i: (i, 0)),
            pl.BlockSpec((8, N), lambda i: (0, 0)),
        ],
        out_specs=pl.BlockSpec((16, Bq), lambda i: (0, i)),
        out_shape=jax.ShapeDtypeStruct((16, N), jnp.int32),
    )(pos8, posT)


def _gather_rows(table, idx):
    V, D = table.shape
    B = idx.shape[0]
    NW = 32
    bpw = B // NW
    c = 128
    nch = bpw // c
    mesh = plsc.VectorSubcoreMesh(core_axis_name="c", subcore_axis_name="s")

    @functools.partial(
        pl.kernel,
        mesh=mesh,
        out_type=jax.ShapeDtypeStruct((B, D), _F32),
        scratch_types=[
            pltpu.VMEM((c,), jnp.int32),
            pltpu.VMEM((c, D), _F32),
            pltpu.SemaphoreType.DMA,
        ],
    )
    def k(tab_hbm, idx_hbm, out_hbm, idx_v, rows_v, sem):
        wid = lax.axis_index("s") * 2 + lax.axis_index("c")
        base = wid * bpw

        def chunk(j, carry):
            off = base + j * c
            pltpu.sync_copy(idx_hbm.at[pl.ds(off, c)], idx_v)
            pltpu.async_copy(tab_hbm.at[idx_v], rows_v, sem).wait()
            pltpu.sync_copy(rows_v, out_hbm.at[pl.ds(off, c)])
            return carry

        lax.fori_loop(0, nch, chunk, 0)

    return k(table, idx)


def _stage0_a(xr, Wm, bm, Wsc, bsc, Win, bin_):
    N = xr.shape[0]
    E = Wsc.shape[1]
    F1 = Win.shape[1]
    B = 512

    def body(x_ref, Wm_ref, bm_ref, Ws_ref, bs_ref, Wi_ref, bi_ref,
             sc_ref, h1_ref):
        xm = _dot(x_ref[...], Wm_ref[...]) + bm_ref[...]
        sc_ref[...] = _dot(xm, Ws_ref[...]) + bs_ref[...]
        h1_ref[...] = _leaky(_dot(xm, Wi_ref[...]) + bi_ref[...])

    full = lambda a: pl.BlockSpec(a.shape, lambda i: (0, 0))
    return pl.pallas_call(
        body,
        grid=(pl.cdiv(N, B),),
        in_specs=[pl.BlockSpec((B, xr.shape[1]), lambda i: (i, 0)),
                  full(Wm), full(bm), full(Wsc), full(bsc), full(Win),
                  full(bin_)],
        out_specs=[pl.BlockSpec((B, E), lambda i: (i, 0)),
                   pl.BlockSpec((B, F1), lambda i: (i, 0))],
        out_shape=[jax.ShapeDtypeStruct((N, E), _F32),
                   jax.ShapeDtypeStruct((N, F1), _F32)],
    )(xr, Wm, bm, Wsc, bsc, Win, bin_)


def _stage_a(x, s, Wsc, bsc, Win, bin_):
    N, I = x.shape
    E = Wsc.shape[1]
    F1 = Win.shape[1]
    B = 512

    def body(x_ref, s_ref, Ws_ref, bs_ref, Wi_ref, bi_ref, sc_ref, h1_ref):
        xm = x_ref[...] * jnp.tanh(s_ref[...])
        sc_ref[...] = _dot(xm, Ws_ref[...]) + bs_ref[...]
        h1_ref[...] = _leaky(_dot(xm, Wi_ref[...]) + bi_ref[...])

    full = lambda a: pl.BlockSpec(a.shape, lambda i: (0, 0))
    return pl.pallas_call(
        body,
        grid=(pl.cdiv(N, B),),
        in_specs=[pl.BlockSpec((B, I), lambda i: (i, 0)),
                  pl.BlockSpec((B, 1), lambda i: (i, 0)),
                  full(Wsc), full(bsc), full(Win), full(bin_)],
        out_specs=[pl.BlockSpec((B, E), lambda i: (i, 0)),
                   pl.BlockSpec((B, F1), lambda i: (i, 0))],
        out_shape=[jax.ShapeDtypeStruct((N, E), _F32),
                   jax.ShapeDtypeStruct((N, F1), _F32)],
    )(x, s, Wsc, bsc, Win, bin_)


def _aggr(tab, g, Wd, Wg, b):
    N, Dp = tab.shape
    Fo = Wd.shape[1]
    B = 512

    def body(t_ref, g_ref, Wd_ref, Wg_ref, b_ref, o_ref):
        dpart = _dot(t_ref[...], Wd_ref[...]) + b_ref[...]
        Wgv = Wg_ref[...]
        acc = None
        for t in range(16):
            v = _leaky(_dot(g_ref[t], Wgv) + dpart)
            acc = v if acc is None else jnp.maximum(acc, v)
        o_ref[...] = acc

    full = lambda a: pl.BlockSpec(a.shape, lambda i: (0, 0))
    return pl.pallas_call(
        body,
        grid=(pl.cdiv(N, B),),
        in_specs=[pl.BlockSpec((B, Dp), lambda i: (i, 0)),
                  pl.BlockSpec((16, B, Dp), lambda i: (0, i, 0)),
                  full(Wd), full(Wg), full(b)],
        out_specs=pl.BlockSpec((B, Fo), lambda i: (i, 0)),
        out_shape=jax.ShapeDtypeStruct((N, Fo), _F32),
    )(tab, g, Wd, Wg, b)


def _stage_b(h3, sc, Wout, bout, pcol, cnorm):
    N, F3 = h3.shape
    E = Wout.shape[1]
    B = 512

    def body(h_ref, s_ref, W_ref, b_ref, p_ref, c_ref, x_ref, sco_ref):
        x = _leaky(_dot(h_ref[...], W_ref[...]) + b_ref[...] + s_ref[...])
        x_ref[...] = x
        sco_ref[...] = _dot(x, p_ref[...]) / c_ref[0, 0]

    full = lambda a: pl.BlockSpec(a.shape, lambda i: (0, 0))
    return pl.pallas_call(
        body,
        grid=(pl.cdiv(N, B),),
        in_specs=[pl.BlockSpec((B, F3), lambda i: (i, 0)),
                  pl.BlockSpec((B, E), lambda i: (i, 0)),
                  full(Wout), full(bout), full(pcol), full(cnorm)],
        out_specs=[pl.BlockSpec((B, E), lambda i: (i, 0)),
                   pl.BlockSpec((B, 1), lambda i: (i, 0))],
        out_shape=[jax.ShapeDtypeStruct((N, E), _F32),
                   jax.ShapeDtypeStruct((N, 1), _F32)],
    )(h3, sc, Wout, bout, pcol, cnorm)


def _head(x, s, Wm, bm, Wc, bc, Wf, bf):
    K = x.shape[0]

    def body(x_ref, s_ref, Wm_ref, bm_ref, Wc_ref, bc_ref, Wf_ref, bf_ref,
             o_ref):
        xm = x_ref[...] * jnp.tanh(s_ref[...])
        h = _leaky(_dot(xm, Wm_ref[...]) + bm_ref[...])
        g = jnp.max(h, axis=0, keepdims=True)
        g = _leaky(_dot(g, Wc_ref[...]) + bc_ref[...])
        o_ref[...] = _dot(g, Wf_ref[...]) + bf_ref[...]

    full = lambda a: pl.BlockSpec(a.shape, lambda i: (0, 0))
    return pl.pallas_call(
        body,
        grid=(1,),
        in_specs=[full(x), full(s), full(Wm), full(bm), full(Wc), full(bc),
                  full(Wf), full(bf)],
        out_specs=pl.BlockSpec((1, 40), lambda i: (0, 0)),
        out_shape=jax.ShapeDtypeStruct((1, 40), _F32),
    )(x, s, Wm, bm, Wc, bc, Wf, bf)


def _pad_cols(a, n):
    return jnp.pad(a, ((0, 0), (0, n - a.shape[1])))


def _pad_to_4096(idx):
    B = int(np.ceil(idx.shape[0] / 4096)) * 4096
    return jnp.pad(idx, (0, B - idx.shape[0]))


def _round128(n):
    return (n + 127) // 128 * 128


def _row(v):
    return v.reshape(1, -1)


def _split_edge_w(W, F):
    W1, W2, W3 = W[:F], W[F:2 * F], W[2 * F:]
    Dp = _round128(F + 3)
    Wd = jnp.concatenate([W1 - W2, -W3], axis=0)
    Wg = jnp.concatenate([W2, W3], axis=0)
    pad = ((0, Dp - (F + 3)), (0, 0))
    return jnp.pad(Wd, pad), jnp.pad(Wg, pad), Dp


def _aggr_step(h, pos, nbr_flat, W, b):
    N, F = h.shape
    Wd, Wg, Dp = _split_edge_w(W, F)
    tab = _pad_cols(jnp.concatenate([h, pos], axis=1), Dp)
    g = _gather_rows(tab, nbr_flat)[: 16 * N].reshape(16, N, Dp)
    return _aggr(tab, g, Wd, Wg, _row(b))


def kernel(x, pos, batch, params):
    xr = x[:, :4, :, :].reshape(x.shape[0], -1)
    ratios = [0.5, 0.5, 0.25, 0.25]
    s = None
    cur_x = xr
    for i, st in enumerate(params["stages"]):
        N = cur_x.shape[0]
        E = st["W_sc"].shape[1]
        pos8 = _pad_cols(pos, 8)
        nbr = _knn16(pos8, pos8.T)
        nbr_flat = _pad_to_4096(nbr.reshape(-1))
        if i == 0:
            sc, h1 = _stage0_a(cur_x, params["W_map"], _row(params["b_map"]),
                               st["W_sc"], _row(st["b_sc"]),
                               st["W_in"], _row(st["b_in"]))
        else:
            sc, h1 = _stage_a(cur_x, s, st["W_sc"], _row(st["b_sc"]),
                              st["W_in"], _row(st["b_in"]))
        h2 = _aggr_step(h1, pos, nbr_flat, st["W_b0"], st["b_b0"])
        h3 = _aggr_step(h2, pos, nbr_flat, st["W_b1"], st["b_b1"])
        cnorm = (jnp.linalg.norm(st["p"]) + 1e-16).reshape(1, 1)
        xs, score = _stage_b(h3, sc, st["W_out"], _row(st["b_out"]),
                             st["p"].reshape(-1, 1), cnorm)
        k = int(np.ceil(ratios[i] * N))
        _, idx = lax.top_k(score[:, 0], k)
        Dp = _round128(E + 4)
        tab = _pad_cols(jnp.concatenate([xs, pos, score], axis=1), Dp)
        rows = _gather_rows(tab, _pad_to_4096(idx))[:k]
        cur_x = rows[:, :E]
        pos = rows[:, E:E + 3]
        s = rows[:, E + 3:E + 4]
    return _head(cur_x, s, params["W_mlp"], _row(params["b_mlp"]),
                 params["W_cls"], _row(params["b_cls"]),
                 params["W_fc"], _row(params["b_fc"]))


# speedup vs baseline: 5.9363x; 1.3842x over previous
import functools

import jax
import jax.numpy as jnp
import numpy as np
from jax import lax
from jax.experimental import pallas as pl
from jax.experimental.pallas import tpu as pltpu
from jax.experimental.pallas import tpu_sc as plsc

_NEG = 0.2
_F32 = jnp.float32


def _leaky(v):
    return jnp.where(v > 0, v, _NEG * v)


def _dot(a, b):
    return jnp.dot(a, b, preferred_element_type=_F32)


def _knn16(pos8, posT):
    N = pos8.shape[0]
    Bq = 256

    def body(p_ref, qt_ref, o_ref):
        P = p_ref[...]
        p2 = jnp.sum(P * P, axis=1, keepdims=True)
        qt = qt_ref[...]
        q2 = jnp.sum(qt * qt, axis=0, keepdims=True)
        d = (q2 - 2.0 * _dot(P, qt)) + p2
        iota0 = lax.broadcasted_iota(jnp.int32, (N, Bq), 0)
        inf = jnp.float32(np.inf)
        for t in range(16):
            am = jnp.argmin(d, axis=0).astype(jnp.int32)
            o_ref[t, :] = am
            d = jnp.where(iota0 == am[None, :], inf, d)

    grid = (pl.cdiv(N, Bq),)
    return pl.pallas_call(
        body,
        grid=grid,
        in_specs=[
            pl.BlockSpec((N, 8), lambda i: (0, 0)),
            pl.BlockSpec((8, Bq), lambda i: (0, i)),
        ],
        out_specs=pl.BlockSpec((16, Bq), lambda i: (0, i)),
        out_shape=jax.ShapeDtypeStruct((16, N), jnp.int32),
    )(pos8, posT)


def _gather_rows(table, idx):
    V, D = table.shape
    B = idx.shape[0]
    NW = 32
    bpw = B // NW
    c = next(cc for cc in (512, 256, 128, 64, 32, 16, 8)
             if bpw % cc == 0 and 2 * cc * D * 4 <= 460 * 1024)
    nch = bpw // c
    npair = nch // 2
    mesh = plsc.VectorSubcoreMesh(core_axis_name="c", subcore_axis_name="s")

    @functools.partial(
        pl.kernel,
        mesh=mesh,
        out_type=jax.ShapeDtypeStruct((B, D), _F32),
        scratch_types=[
            pltpu.VMEM((bpw,), jnp.int32),
            pltpu.VMEM((c, D), _F32),
            pltpu.VMEM((c, D), _F32),
            pltpu.SemaphoreType.DMA,
            pltpu.SemaphoreType.DMA,
        ],
    )
    def k(tab_hbm, idx_hbm, out_hbm, idx_v, r0, r1, s0, s1):
        wid = lax.axis_index("s") * 2 + lax.axis_index("c")
        base = wid * bpw
        pltpu.sync_copy(idx_hbm.at[pl.ds(base, bpw)], idx_v)

        def gather(j, buf, sem):
            return pltpu.async_copy(
                tab_hbm.at[idx_v.at[pl.ds(j * c, c)]], buf, sem)

        def put(j, buf):
            pltpu.sync_copy(buf, out_hbm.at[pl.ds(base + j * c, c)])

        def pair(i2, carry):
            j0 = 2 * i2
            cp0 = gather(j0, r0, s0)
            cp1 = gather(j0 + 1, r1, s1)
            cp0.wait()
            put(j0, r0)
            cp1.wait()
            put(j0 + 1, r1)
            return carry

        lax.fori_loop(0, npair, pair, 0)
        if nch % 2:
            cp = gather(nch - 1, r0, s0)
            cp.wait()
            put(nch - 1, r0)

    return k(table, idx)


def _stage0_a(xr, Wm, bm, Wsc, bsc, Win, bin_):
    N = xr.shape[0]
    E = Wsc.shape[1]
    F1 = Win.shape[1]
    B = 512

    def body(x_ref, Wm_ref, bm_ref, Ws_ref, bs_ref, Wi_ref, bi_ref,
             sc_ref, h1_ref):
        xm = _dot(x_ref[...], Wm_ref[...]) + bm_ref[...]
        sc_ref[...] = _dot(xm, Ws_ref[...]) + bs_ref[...]
        h1_ref[...] = _leaky(_dot(xm, Wi_ref[...]) + bi_ref[...])

    full = lambda a: pl.BlockSpec(a.shape, lambda i: (0, 0))
    return pl.pallas_call(
        body,
        grid=(pl.cdiv(N, B),),
        in_specs=[pl.BlockSpec((B, xr.shape[1]), lambda i: (i, 0)),
                  full(Wm), full(bm), full(Wsc), full(bsc), full(Win),
                  full(bin_)],
        out_specs=[pl.BlockSpec((B, E), lambda i: (i, 0)),
                   pl.BlockSpec((B, F1), lambda i: (i, 0))],
        out_shape=[jax.ShapeDtypeStruct((N, E), _F32),
                   jax.ShapeDtypeStruct((N, F1), _F32)],
    )(xr, Wm, bm, Wsc, bsc, Win, bin_)


def _stage_a(x, s, Wsc, bsc, Win, bin_):
    N, I = x.shape
    E = Wsc.shape[1]
    F1 = Win.shape[1]
    B = 512

    def body(x_ref, s_ref, Ws_ref, bs_ref, Wi_ref, bi_ref, sc_ref, h1_ref):
        xm = x_ref[...] * jnp.tanh(s_ref[...])
        sc_ref[...] = _dot(xm, Ws_ref[...]) + bs_ref[...]
        h1_ref[...] = _leaky(_dot(xm, Wi_ref[...]) + bi_ref[...])

    full = lambda a: pl.BlockSpec(a.shape, lambda i: (0, 0))
    return pl.pallas_call(
        body,
        grid=(pl.cdiv(N, B),),
        in_specs=[pl.BlockSpec((B, I), lambda i: (i, 0)),
                  pl.BlockSpec((B, 1), lambda i: (i, 0)),
                  full(Wsc), full(bsc), full(Win), full(bin_)],
        out_specs=[pl.BlockSpec((B, E), lambda i: (i, 0)),
                   pl.BlockSpec((B, F1), lambda i: (i, 0))],
        out_shape=[jax.ShapeDtypeStruct((N, E), _F32),
                   jax.ShapeDtypeStruct((N, F1), _F32)],
    )(x, s, Wsc, bsc, Win, bin_)


def _aggr(tab, g, Wd, Wg, b):
    N, Dp = tab.shape
    Fo = Wd.shape[1]
    B = 512

    def body(t_ref, g_ref, Wd_ref, Wg_ref, b_ref, o_ref):
        dpart = _dot(t_ref[...], Wd_ref[...]) + b_ref[...]
        Wgv = Wg_ref[...]
        acc = None
        for t in range(16):
            v = _leaky(_dot(g_ref[t], Wgv) + dpart)
            acc = v if acc is None else jnp.maximum(acc, v)
        o_ref[...] = acc

    full = lambda a: pl.BlockSpec(a.shape, lambda i: (0, 0))
    return pl.pallas_call(
        body,
        grid=(pl.cdiv(N, B),),
        in_specs=[pl.BlockSpec((B, Dp), lambda i: (i, 0)),
                  pl.BlockSpec((16, B, Dp), lambda i: (0, i, 0)),
                  full(Wd), full(Wg), full(b)],
        out_specs=pl.BlockSpec((B, Fo), lambda i: (i, 0)),
        out_shape=jax.ShapeDtypeStruct((N, Fo), _F32),
    )(tab, g, Wd, Wg, b)


def _stage_b(h3, sc, Wout, bout, pcol, cnorm):
    N, F3 = h3.shape
    E = Wout.shape[1]
    B = 512

    def body(h_ref, s_ref, W_ref, b_ref, p_ref, c_ref, x_ref, sco_ref):
        x = _leaky(_dot(h_ref[...], W_ref[...]) + b_ref[...] + s_ref[...])
        x_ref[...] = x
        sco_ref[...] = _dot(x, p_ref[...]) / c_ref[0, 0]

    full = lambda a: pl.BlockSpec(a.shape, lambda i: (0, 0))
    return pl.pallas_call(
        body,
        grid=(pl.cdiv(N, B),),
        in_specs=[pl.BlockSpec((B, F3), lambda i: (i, 0)),
                  pl.BlockSpec((B, E), lambda i: (i, 0)),
                  full(Wout), full(bout), full(pcol), full(cnorm)],
        out_specs=[pl.BlockSpec((B, E), lambda i: (i, 0)),
                   pl.BlockSpec((B, 1), lambda i: (i, 0))],
        out_shape=[jax.ShapeDtypeStruct((N, E), _F32),
                   jax.ShapeDtypeStruct((N, 1), _F32)],
    )(h3, sc, Wout, bout, pcol, cnorm)


def _head(x, s, Wm, bm, Wc, bc, Wf, bf):
    K = x.shape[0]

    def body(x_ref, s_ref, Wm_ref, bm_ref, Wc_ref, bc_ref, Wf_ref, bf_ref,
             o_ref):
        xm = x_ref[...] * jnp.tanh(s_ref[...])
        h = _leaky(_dot(xm, Wm_ref[...]) + bm_ref[...])
        g = jnp.max(h, axis=0, keepdims=True)
        g = _leaky(_dot(g, Wc_ref[...]) + bc_ref[...])
        o_ref[...] = _dot(g, Wf_ref[...]) + bf_ref[...]

    full = lambda a: pl.BlockSpec(a.shape, lambda i: (0, 0))
    return pl.pallas_call(
        body,
        grid=(1,),
        in_specs=[full(x), full(s), full(Wm), full(bm), full(Wc), full(bc),
                  full(Wf), full(bf)],
        out_specs=pl.BlockSpec((1, 40), lambda i: (0, 0)),
        out_shape=jax.ShapeDtypeStruct((1, 40), _F32),
    )(x, s, Wm, bm, Wc, bc, Wf, bf)


def _pad_cols(a, n):
    return jnp.pad(a, ((0, 0), (0, n - a.shape[1])))


def _pad_to_4096(idx):
    B = int(np.ceil(idx.shape[0] / 4096)) * 4096
    return jnp.pad(idx, (0, B - idx.shape[0]))


def _round128(n):
    return (n + 127) // 128 * 128


def _row(v):
    return v.reshape(1, -1)


def _split_edge_w(W, F):
    W1, W2, W3 = W[:F], W[F:2 * F], W[2 * F:]
    Dp = _round128(F + 3)
    Wd = jnp.concatenate([W1 - W2, -W3], axis=0)
    Wg = jnp.concatenate([W2, W3], axis=0)
    pad = ((0, Dp - (F + 3)), (0, 0))
    return jnp.pad(Wd, pad), jnp.pad(Wg, pad), Dp


def _aggr_step(h, pos, nbr_flat, W, b):
    N, F = h.shape
    Wd, Wg, Dp = _split_edge_w(W, F)
    tab = _pad_cols(jnp.concatenate([h, pos], axis=1), Dp)
    g = _gather_rows(tab, nbr_flat)[: 16 * N].reshape(16, N, Dp)
    return _aggr(tab, g, Wd, Wg, _row(b))


def kernel(x, pos, batch, params):
    xr = x[:, :4, :, :].reshape(x.shape[0], -1)
    ratios = [0.5, 0.5, 0.25, 0.25]
    s = None
    cur_x = xr
    for i, st in enumerate(params["stages"]):
        N = cur_x.shape[0]
        E = st["W_sc"].shape[1]
        pos8 = _pad_cols(pos, 8)
        nbr = _knn16(pos8, pos8.T)
        nbr_flat = _pad_to_4096(nbr.reshape(-1))
        if i == 0:
            sc, h1 = _stage0_a(cur_x, params["W_map"], _row(params["b_map"]),
                               st["W_sc"], _row(st["b_sc"]),
                               st["W_in"], _row(st["b_in"]))
        else:
            sc, h1 = _stage_a(cur_x, s, st["W_sc"], _row(st["b_sc"]),
                              st["W_in"], _row(st["b_in"]))
        h2 = _aggr_step(h1, pos, nbr_flat, st["W_b0"], st["b_b0"])
        h3 = _aggr_step(h2, pos, nbr_flat, st["W_b1"], st["b_b1"])
        cnorm = (jnp.linalg.norm(st["p"]) + 1e-16).reshape(1, 1)
        xs, score = _stage_b(h3, sc, st["W_out"], _row(st["b_out"]),
                             st["p"].reshape(-1, 1), cnorm)
        k = int(np.ceil(ratios[i] * N))
        _, idx = lax.top_k(score[:, 0], k)
        Dp = _round128(E + 4)
        tab = _pad_cols(jnp.concatenate([xs, pos, score], axis=1), Dp)
        rows = _gather_rows(tab, _pad_to_4096(idx))[:k]
        cur_x = rows[:, :E]
        pos = rows[:, E:E + 3]
        s = rows[:, E + 3:E + 4]
    return _head(cur_x, s, params["W_mlp"], _row(params["b_mlp"]),
                 params["W_cls"], _row(params["b_cls"]),
                 params["W_fc"], _row(params["b_fc"]))


# speedup vs baseline: 6.9029x; 1.1628x over previous
import functools

import jax
import jax.numpy as jnp
import numpy as np
from jax import lax
from jax.experimental import pallas as pl
from jax.experimental.pallas import tpu as pltpu
from jax.experimental.pallas import tpu_sc as plsc

_NEG = 0.2
_F32 = jnp.float32


def _leaky(v):
    return jnp.where(v > 0, v, _NEG * v)


def _dot(a, b):
    return jnp.dot(a, b, preferred_element_type=_F32)


def _knn16(pos8, posT):
    N = pos8.shape[0]
    Bq = 256

    def body(p_ref, qt_ref, o_ref):
        P = p_ref[...]
        p2 = jnp.sum(P * P, axis=1, keepdims=True)
        qt = qt_ref[...]
        q2 = jnp.sum(qt * qt, axis=0, keepdims=True)
        d = (q2 - 2.0 * _dot(P, qt)) + p2
        iota0 = lax.broadcasted_iota(jnp.int32, (N, Bq), 0)
        inf = jnp.float32(np.inf)
        for t in range(16):
            am = jnp.argmin(d, axis=0).astype(jnp.int32)
            o_ref[t, :] = am
            d = jnp.where(iota0 == am[None, :], inf, d)

    grid = (pl.cdiv(N, Bq),)
    return pl.pallas_call(
        body,
        grid=grid,
        in_specs=[
            pl.BlockSpec((N, 8), lambda i: (0, 0)),
            pl.BlockSpec((8, Bq), lambda i: (0, i)),
        ],
        out_specs=pl.BlockSpec((16, Bq), lambda i: (0, i)),
        out_shape=jax.ShapeDtypeStruct((16, N), jnp.int32),
    )(pos8, posT)


def _gather_rows(table, idx):
    V, D = table.shape
    B = idx.shape[0]
    NW = 32
    bpw = B // NW
    c = next(cc for cc in (512, 256, 128, 64, 32, 16, 8)
             if bpw % cc == 0 and 2 * cc * D * 4 <= 460 * 1024)
    nch = bpw // c
    npair = nch // 2
    mesh = plsc.VectorSubcoreMesh(core_axis_name="c", subcore_axis_name="s")

    @functools.partial(
        pl.kernel,
        mesh=mesh,
        out_type=jax.ShapeDtypeStruct((B, D), _F32),
        scratch_types=[
            pltpu.VMEM((bpw,), jnp.int32),
            pltpu.VMEM((c, D), _F32),
            pltpu.VMEM((c, D), _F32),
            pltpu.SemaphoreType.DMA,
            pltpu.SemaphoreType.DMA,
        ],
    )
    def k(tab_hbm, idx_hbm, out_hbm, idx_v, r0, r1, s0, s1):
        wid = lax.axis_index("s") * 2 + lax.axis_index("c")
        base = wid * bpw
        pltpu.sync_copy(idx_hbm.at[pl.ds(base, bpw)], idx_v)

        def gather(j, buf, sem):
            return pltpu.async_copy(
                tab_hbm.at[idx_v.at[pl.ds(j * c, c)]], buf, sem)

        def put(j, buf):
            pltpu.sync_copy(buf, out_hbm.at[pl.ds(base + j * c, c)])

        def pair(i2, carry):
            j0 = 2 * i2
            cp0 = gather(j0, r0, s0)
            cp1 = gather(j0 + 1, r1, s1)
            cp0.wait()
            put(j0, r0)
            cp1.wait()
            put(j0 + 1, r1)
            return carry

        lax.fori_loop(0, npair, pair, 0)
        if nch % 2:
            cp = gather(nch - 1, r0, s0)
            cp.wait()
            put(nch - 1, r0)

    return k(table, idx)


def _tc_gather(table, idx):
    V, D = table.shape
    K = idx.shape[0]
    B = 256

    def body(i_ref, t_ref, o_ref):
        ii = i_ref[...]
        iota = lax.broadcasted_iota(jnp.int32, (B, V), 1)
        oh = (iota == ii).astype(_F32)
        o_ref[...] = _dot(oh, t_ref[...])

    return pl.pallas_call(
        body,
        grid=(pl.cdiv(K, B),),
        in_specs=[pl.BlockSpec((B, 1), lambda i: (i, 0)),
                  pl.BlockSpec((V, D), lambda i: (0, 0))],
        out_specs=pl.BlockSpec((B, D), lambda i: (i, 0)),
        out_shape=jax.ShapeDtypeStruct((K, D), _F32),
    )(idx.reshape(-1, 1), table)


def _stage0_a(xr, Wm, bm, Wsc, bsc, Win, bin_):
    N = xr.shape[0]
    E = Wsc.shape[1]
    F1 = Win.shape[1]
    B = 512

    def body(x_ref, Wm_ref, bm_ref, Ws_ref, bs_ref, Wi_ref, bi_ref,
             sc_ref, h1_ref):
        xm = _dot(x_ref[...], Wm_ref[...]) + bm_ref[...]
        sc_ref[...] = _dot(xm, Ws_ref[...]) + bs_ref[...]
        h1_ref[...] = _leaky(_dot(xm, Wi_ref[...]) + bi_ref[...])

    full = lambda a: pl.BlockSpec(a.shape, lambda i: (0, 0))
    return pl.pallas_call(
        body,
        grid=(pl.cdiv(N, B),),
        in_specs=[pl.BlockSpec((B, xr.shape[1]), lambda i: (i, 0)),
                  full(Wm), full(bm), full(Wsc), full(bsc), full(Win),
                  full(bin_)],
        out_specs=[pl.BlockSpec((B, E), lambda i: (i, 0)),
                   pl.BlockSpec((B, F1), lambda i: (i, 0))],
        out_shape=[jax.ShapeDtypeStruct((N, E), _F32),
                   jax.ShapeDtypeStruct((N, F1), _F32)],
    )(xr, Wm, bm, Wsc, bsc, Win, bin_)


def _stage_a(x, s, Wsc, bsc, Win, bin_):
    N, I = x.shape
    E = Wsc.shape[1]
    F1 = Win.shape[1]
    B = 512

    def body(x_ref, s_ref, Ws_ref, bs_ref, Wi_ref, bi_ref, sc_ref, h1_ref):
        xm = x_ref[...] * jnp.tanh(s_ref[...])
        sc_ref[...] = _dot(xm, Ws_ref[...]) + bs_ref[...]
        h1_ref[...] = _leaky(_dot(xm, Wi_ref[...]) + bi_ref[...])

    full = lambda a: pl.BlockSpec(a.shape, lambda i: (0, 0))
    return pl.pallas_call(
        body,
        grid=(pl.cdiv(N, B),),
        in_specs=[pl.BlockSpec((B, I), lambda i: (i, 0)),
                  pl.BlockSpec((B, 1), lambda i: (i, 0)),
                  full(Wsc), full(bsc), full(Win), full(bin_)],
        out_specs=[pl.BlockSpec((B, E), lambda i: (i, 0)),
                   pl.BlockSpec((B, F1), lambda i: (i, 0))],
        out_shape=[jax.ShapeDtypeStruct((N, E), _F32),
                   jax.ShapeDtypeStruct((N, F1), _F32)],
    )(x, s, Wsc, bsc, Win, bin_)


def _aggr(tab, g, Wd, Wg, b):
    N, Dp = tab.shape
    Fo = Wd.shape[1]
    B = 512

    def body(t_ref, g_ref, Wd_ref, Wg_ref, b_ref, o_ref):
        dpart = _dot(t_ref[...], Wd_ref[...]) + b_ref[...]
        Wgv = Wg_ref[...]
        acc = None
        for t in range(16):
            v = _leaky(_dot(g_ref[t], Wgv) + dpart)
            acc = v if acc is None else jnp.maximum(acc, v)
        o_ref[...] = acc

    full = lambda a: pl.BlockSpec(a.shape, lambda i: (0, 0))
    return pl.pallas_call(
        body,
        grid=(pl.cdiv(N, B),),
        in_specs=[pl.BlockSpec((B, Dp), lambda i: (i, 0)),
                  pl.BlockSpec((16, B, Dp), lambda i: (0, i, 0)),
                  full(Wd), full(Wg), full(b)],
        out_specs=pl.BlockSpec((B, Fo), lambda i: (i, 0)),
        out_shape=jax.ShapeDtypeStruct((N, Fo), _F32),
    )(tab, g, Wd, Wg, b)


def _stage_b(h3, sc, Wout, bout, pcol, cnorm):
    N, F3 = h3.shape
    E = Wout.shape[1]
    B = 512

    def body(h_ref, s_ref, W_ref, b_ref, p_ref, c_ref, x_ref, sco_ref):
        x = _leaky(_dot(h_ref[...], W_ref[...]) + b_ref[...] + s_ref[...])
        x_ref[...] = x
        sco_ref[...] = _dot(x, p_ref[...]) / c_ref[0, 0]

    full = lambda a: pl.BlockSpec(a.shape, lambda i: (0, 0))
    return pl.pallas_call(
        body,
        grid=(pl.cdiv(N, B),),
        in_specs=[pl.BlockSpec((B, F3), lambda i: (i, 0)),
                  pl.BlockSpec((B, E), lambda i: (i, 0)),
                  full(Wout), full(bout), full(pcol), full(cnorm)],
        out_specs=[pl.BlockSpec((B, E), lambda i: (i, 0)),
                   pl.BlockSpec((B, 1), lambda i: (i, 0))],
        out_shape=[jax.ShapeDtypeStruct((N, E), _F32),
                   jax.ShapeDtypeStruct((N, 1), _F32)],
    )(h3, sc, Wout, bout, pcol, cnorm)


def _head(x, s, Wm, bm, Wc, bc, Wf, bf):
    K = x.shape[0]

    def body(x_ref, s_ref, Wm_ref, bm_ref, Wc_ref, bc_ref, Wf_ref, bf_ref,
             o_ref):
        xm = x_ref[...] * jnp.tanh(s_ref[...])
        h = _leaky(_dot(xm, Wm_ref[...]) + bm_ref[...])
        g = jnp.max(h, axis=0, keepdims=True)
        g = _leaky(_dot(g, Wc_ref[...]) + bc_ref[...])
        o_ref[...] = _dot(g, Wf_ref[...]) + bf_ref[...]

    full = lambda a: pl.BlockSpec(a.shape, lambda i: (0, 0))
    return pl.pallas_call(
        body,
        grid=(1,),
        in_specs=[full(x), full(s), full(Wm), full(bm), full(Wc), full(bc),
                  full(Wf), full(bf)],
        out_specs=pl.BlockSpec((1, 40), lambda i: (0, 0)),
        out_shape=jax.ShapeDtypeStruct((1, 40), _F32),
    )(x, s, Wm, bm, Wc, bc, Wf, bf)


def _pad_cols(a, n):
    return jnp.pad(a, ((0, 0), (0, n - a.shape[1])))


def _pad_to_4096(idx):
    B = int(np.ceil(idx.shape[0] / 4096)) * 4096
    return jnp.pad(idx, (0, B - idx.shape[0]))


def _round128(n):
    return (n + 127) // 128 * 128


def _row(v):
    return v.reshape(1, -1)


def _split_edge_w(W, F, Dp):
    W1, W2, W3 = W[:F], W[F:2 * F], W[2 * F:]
    Wd = jnp.concatenate([W1 - W2, -W3], axis=0)
    Wg = jnp.concatenate([W2, W3], axis=0)
    pad = ((0, Dp - (F + 3)), (0, 0))
    return jnp.pad(Wd, pad), jnp.pad(Wg, pad)


def _aggr_step(h, pos, nbr, W, b, use_sc):
    N, F = h.shape
    if use_sc:
        Dp = _round128(F + 3)
        tab = _pad_cols(jnp.concatenate([h, pos], axis=1), Dp)
        g = _gather_rows(tab, _pad_to_4096(nbr.reshape(-1)))
    else:
        Dp = (F + 3 + 7) // 8 * 8
        tab = _pad_cols(jnp.concatenate([h, pos], axis=1), Dp)
        g = _tc_gather(tab, nbr.reshape(-1))
    Wd, Wg = _split_edge_w(W, F, Dp)
    g = g[: 16 * N].reshape(16, N, Dp)
    return _aggr(tab, g, Wd, Wg, _row(b))


def kernel(x, pos, batch, params):
    xr = x[:, :4, :, :].reshape(x.shape[0], -1)
    ratios = [0.5, 0.5, 0.25, 0.25]
    s = None
    cur_x = xr
    for i, st in enumerate(params["stages"]):
        N = cur_x.shape[0]
        E = st["W_sc"].shape[1]
        pos8 = _pad_cols(pos, 8)
        nbr = _knn16(pos8, pos8.T)
        use_sc = i < 2
        if i == 0:
            sc, h1 = _stage0_a(cur_x, params["W_map"], _row(params["b_map"]),
                               st["W_sc"], _row(st["b_sc"]),
                               st["W_in"], _row(st["b_in"]))
        else:
            sc, h1 = _stage_a(cur_x, s, st["W_sc"], _row(st["b_sc"]),
                              st["W_in"], _row(st["b_in"]))
        h2 = _aggr_step(h1, pos, nbr, st["W_b0"], st["b_b0"], use_sc)
        h3 = _aggr_step(h2, pos, nbr, st["W_b1"], st["b_b1"], use_sc)
        cnorm = (jnp.linalg.norm(st["p"]) + 1e-16).reshape(1, 1)
        xs, score = _stage_b(h3, sc, st["W_out"], _row(st["b_out"]),
                             st["p"].reshape(-1, 1), cnorm)
        k = int(np.ceil(ratios[i] * N))
        _, idx = lax.top_k(score[:, 0], k)
        tab = jnp.concatenate([xs, pos, score], axis=1)
        rows = _tc_gather(tab, idx)
        cur_x = rows[:, :E]
        pos = rows[:, E:E + 3]
        s = rows[:, E + 3:E + 4]
    return _head(cur_x, s, params["W_mlp"], _row(params["b_mlp"]),
                 params["W_cls"], _row(params["b_cls"]),
                 params["W_fc"], _row(params["b_fc"]))


# speedup vs baseline: 7.4911x; 1.0852x over previous
import functools

import jax
import jax.numpy as jnp
import numpy as np
from jax import lax
from jax.experimental import pallas as pl
from jax.experimental.pallas import tpu as pltpu
from jax.experimental.pallas import tpu_sc as plsc

_NEG = 0.2
_F32 = jnp.float32


def _leaky(v):
    return jnp.where(v > 0, v, _NEG * v)


def _dot(a, b):
    return jnp.dot(a, b, preferred_element_type=_F32)


def _knn16(pos8, posT):
    N = pos8.shape[0]
    Bq = 256

    def body(p_ref, qt_ref, o_ref):
        P = p_ref[...]
        p2 = jnp.sum(P * P, axis=1, keepdims=True)
        qt = qt_ref[...]
        q2 = jnp.sum(qt * qt, axis=0, keepdims=True)
        d = (q2 - 2.0 * _dot(P, qt)) + p2
        iota0 = lax.broadcasted_iota(jnp.int32, (N, Bq), 0)
        inf = jnp.float32(np.inf)
        for t in range(16):
            am = jnp.argmin(d, axis=0).astype(jnp.int32)
            o_ref[t, :] = am
            d = jnp.where(iota0 == am[None, :], inf, d)

    grid = (pl.cdiv(N, Bq),)
    return pl.pallas_call(
        body,
        grid=grid,
        in_specs=[
            pl.BlockSpec((N, 8), lambda i: (0, 0)),
            pl.BlockSpec((8, Bq), lambda i: (0, i)),
        ],
        out_specs=pl.BlockSpec((16, Bq), lambda i: (0, i)),
        out_shape=jax.ShapeDtypeStruct((16, N), jnp.int32),
    )(pos8, posT)


def _gather_rows(table, idx):
    V, D = table.shape
    B = idx.shape[0]
    NW = 32
    bpw = B // NW
    c = next(cc for cc in (512, 256, 128, 64, 32, 16, 8)
             if bpw % cc == 0 and 2 * cc * D * 4 <= 460 * 1024)
    nch = bpw // c
    npair = nch // 2
    mesh = plsc.VectorSubcoreMesh(core_axis_name="c", subcore_axis_name="s")

    @functools.partial(
        pl.kernel,
        mesh=mesh,
        out_type=jax.ShapeDtypeStruct((B, D), _F32),
        scratch_types=[
            pltpu.VMEM((bpw,), jnp.int32),
            pltpu.VMEM((c, D), _F32),
            pltpu.VMEM((c, D), _F32),
            pltpu.SemaphoreType.DMA,
            pltpu.SemaphoreType.DMA,
        ],
    )
    def k(tab_hbm, idx_hbm, out_hbm, idx_v, r0, r1, s0, s1):
        wid = lax.axis_index("s") * 2 + lax.axis_index("c")
        base = wid * bpw
        pltpu.sync_copy(idx_hbm.at[pl.ds(base, bpw)], idx_v)

        def gather(j, buf, sem):
            return pltpu.async_copy(
                tab_hbm.at[idx_v.at[pl.ds(j * c, c)]], buf, sem)

        def put(j, buf):
            pltpu.sync_copy(buf, out_hbm.at[pl.ds(base + j * c, c)])

        def pair(i2, carry):
            j0 = 2 * i2
            cp0 = gather(j0, r0, s0)
            cp1 = gather(j0 + 1, r1, s1)
            cp0.wait()
            put(j0, r0)
            cp1.wait()
            put(j0 + 1, r1)
            return carry

        lax.fori_loop(0, npair, pair, 0)
        if nch % 2:
            cp = gather(nch - 1, r0, s0)
            cp.wait()
            put(nch - 1, r0)

    return k(table, idx)


def _tc_gather(table, idx):
    V, D = table.shape
    K = idx.shape[0]
    B = 256

    def body(i_ref, t_ref, o_ref):
        ii = i_ref[...]
        iota = lax.broadcasted_iota(jnp.int32, (B, V), 1)
        oh = (iota == ii).astype(_F32)
        o_ref[...] = _dot(oh, t_ref[...])

    return pl.pallas_call(
        body,
        grid=(pl.cdiv(K, B),),
        in_specs=[pl.BlockSpec((B, 1), lambda i: (i, 0)),
                  pl.BlockSpec((V, D), lambda i: (0, 0))],
        out_specs=pl.BlockSpec((B, D), lambda i: (i, 0)),
        out_shape=jax.ShapeDtypeStruct((K, D), _F32),
    )(idx.reshape(-1, 1), table)


def _stage0_a(xr, Wm, bm, Wsc, bsc, Win, bin_):
    N = xr.shape[0]
    E = Wsc.shape[1]
    F1 = Win.shape[1]
    B = 512

    def body(x_ref, Wm_ref, bm_ref, Ws_ref, bs_ref, Wi_ref, bi_ref,
             sc_ref, h1_ref):
        xm = _dot(x_ref[...], Wm_ref[...]) + bm_ref[...]
        sc_ref[...] = _dot(xm, Ws_ref[...]) + bs_ref[...]
        h1_ref[...] = _leaky(_dot(xm, Wi_ref[...]) + bi_ref[...])

    full = lambda a: pl.BlockSpec(a.shape, lambda i: (0, 0))
    return pl.pallas_call(
        body,
        grid=(pl.cdiv(N, B),),
        in_specs=[pl.BlockSpec((B, xr.shape[1]), lambda i: (i, 0)),
                  full(Wm), full(bm), full(Wsc), full(bsc), full(Win),
                  full(bin_)],
        out_specs=[pl.BlockSpec((B, E), lambda i: (i, 0)),
                   pl.BlockSpec((B, F1), lambda i: (i, 0))],
        out_shape=[jax.ShapeDtypeStruct((N, E), _F32),
                   jax.ShapeDtypeStruct((N, F1), _F32)],
    )(xr, Wm, bm, Wsc, bsc, Win, bin_)


def _stage_a(x, s, Wsc, bsc, Win, bin_):
    N, I = x.shape
    E = Wsc.shape[1]
    F1 = Win.shape[1]
    B = 512

    def body(x_ref, s_ref, Ws_ref, bs_ref, Wi_ref, bi_ref, sc_ref, h1_ref):
        xm = x_ref[...] * jnp.tanh(s_ref[...])
        sc_ref[...] = _dot(xm, Ws_ref[...]) + bs_ref[...]
        h1_ref[...] = _leaky(_dot(xm, Wi_ref[...]) + bi_ref[...])

    full = lambda a: pl.BlockSpec(a.shape, lambda i: (0, 0))
    return pl.pallas_call(
        body,
        grid=(pl.cdiv(N, B),),
        in_specs=[pl.BlockSpec((B, I), lambda i: (i, 0)),
                  pl.BlockSpec((B, 1), lambda i: (i, 0)),
                  full(Wsc), full(bsc), full(Win), full(bin_)],
        out_specs=[pl.BlockSpec((B, E), lambda i: (i, 0)),
                   pl.BlockSpec((B, F1), lambda i: (i, 0))],
        out_shape=[jax.ShapeDtypeStruct((N, E), _F32),
                   jax.ShapeDtypeStruct((N, F1), _F32)],
    )(x, s, Wsc, bsc, Win, bin_)


def _aggr(tab, g, Wd, Wg, b):
    N, Dp = tab.shape
    Ds = g.shape[2]
    Fo = Wd.shape[1]
    B = 512

    def body(t_ref, g_ref, Wd_ref, Wg_ref, b_ref, o_ref):
        dpart = _dot(t_ref[...], Wd_ref[...]) + b_ref[...]
        Wgv = Wg_ref[...]
        acc = None
        for t in range(16):
            v = _leaky(_dot(g_ref[t], Wgv) + dpart)
            acc = v if acc is None else jnp.maximum(acc, v)
        o_ref[...] = acc

    full = lambda a: pl.BlockSpec(a.shape, lambda i: (0, 0))
    return pl.pallas_call(
        body,
        grid=(pl.cdiv(N, B),),
        in_specs=[pl.BlockSpec((B, Dp), lambda i: (i, 0)),
                  pl.BlockSpec((16, B, Ds), lambda i: (0, i, 0)),
                  full(Wd), full(Wg), full(b)],
        out_specs=pl.BlockSpec((B, Fo), lambda i: (i, 0)),
        out_shape=jax.ShapeDtypeStruct((N, Fo), _F32),
    )(tab, g, Wd, Wg, b)


def _stage_b(h3, sc, Wout, bout, pcol, cnorm):
    N, F3 = h3.shape
    E = Wout.shape[1]
    B = 512

    def body(h_ref, s_ref, W_ref, b_ref, p_ref, c_ref, x_ref, sco_ref):
        x = _leaky(_dot(h_ref[...], W_ref[...]) + b_ref[...] + s_ref[...])
        x_ref[...] = x
        sco_ref[...] = _dot(x, p_ref[...]) / c_ref[0, 0]

    full = lambda a: pl.BlockSpec(a.shape, lambda i: (0, 0))
    return pl.pallas_call(
        body,
        grid=(pl.cdiv(N, B),),
        in_specs=[pl.BlockSpec((B, F3), lambda i: (i, 0)),
                  pl.BlockSpec((B, E), lambda i: (i, 0)),
                  full(Wout), full(bout), full(pcol), full(cnorm)],
        out_specs=[pl.BlockSpec((B, E), lambda i: (i, 0)),
                   pl.BlockSpec((B, 1), lambda i: (i, 0))],
        out_shape=[jax.ShapeDtypeStruct((N, E), _F32),
                   jax.ShapeDtypeStruct((N, 1), _F32)],
    )(h3, sc, Wout, bout, pcol, cnorm)


def _head(x, s, Wm, bm, Wc, bc, Wf, bf):
    K = x.shape[0]

    def body(x_ref, s_ref, Wm_ref, bm_ref, Wc_ref, bc_ref, Wf_ref, bf_ref,
             o_ref):
        xm = x_ref[...] * jnp.tanh(s_ref[...])
        h = _leaky(_dot(xm, Wm_ref[...]) + bm_ref[...])
        g = jnp.max(h, axis=0, keepdims=True)
        g = _leaky(_dot(g, Wc_ref[...]) + bc_ref[...])
        o_ref[...] = _dot(g, Wf_ref[...]) + bf_ref[...]

    full = lambda a: pl.BlockSpec(a.shape, lambda i: (0, 0))
    return pl.pallas_call(
        body,
        grid=(1,),
        in_specs=[full(x), full(s), full(Wm), full(bm), full(Wc), full(bc),
                  full(Wf), full(bf)],
        out_specs=pl.BlockSpec((1, 40), lambda i: (0, 0)),
        out_shape=jax.ShapeDtypeStruct((1, 40), _F32),
    )(x, s, Wm, bm, Wc, bc, Wf, bf)


def _pad_cols(a, n):
    return jnp.pad(a, ((0, 0), (0, n - a.shape[1])))


def _pad_to_4096(idx):
    B = int(np.ceil(idx.shape[0] / 4096)) * 4096
    return jnp.pad(idx, (0, B - idx.shape[0]))


def _round128(n):
    return (n + 127) // 128 * 128


def _row(v):
    return v.reshape(1, -1)


def _split_edge_w(W, F, Dp):
    W1, W2, W3 = W[:F], W[F:2 * F], W[2 * F:]
    Wd = jnp.concatenate([W1 - W2, -W3], axis=0)
    Wg = jnp.concatenate([W2, W3], axis=0)
    pad = ((0, Dp - (F + 3)), (0, 0))
    return jnp.pad(Wd, pad), jnp.pad(Wg, pad)


def _aggr_step(h, pos, nbr, W, b, use_sc):
    N, F = h.shape
    Npad = nbr.shape[1]
    if use_sc:
        Dp = _round128(F + 3)
        tab = _pad_cols(jnp.concatenate([h, pos], axis=1), Dp)
        g = _gather_rows(tab, nbr.reshape(-1))
    else:
        Dp = (F + 3 + 7) // 8 * 8
        tab = _pad_cols(jnp.concatenate([h, pos], axis=1), Dp)
        g = _tc_gather(tab, nbr.reshape(-1))
    Wd, Wg = _split_edge_w(W, F, Dp)
    g = g.reshape(16, Npad, Dp)
    return _aggr(tab, g, Wd, Wg, _row(b))


def kernel(x, pos, batch, params):
    xr = x[:, :4, :, :].reshape(x.shape[0], -1)
    ratios = [0.5, 0.5, 0.25, 0.25]
    s = None
    cur_x = xr
    for i, st in enumerate(params["stages"]):
        N = cur_x.shape[0]
        E = st["W_sc"].shape[1]
        pos8 = _pad_cols(pos, 8)
        nbr = _knn16(pos8, pos8.T)
        use_sc = i < 2
        if use_sc:
            Npad = (N + 255) // 256 * 256
            nbr = jnp.pad(nbr, ((0, 0), (0, Npad - N)))
        if i == 0:
            sc, h1 = _stage0_a(cur_x, params["W_map"], _row(params["b_map"]),
                               st["W_sc"], _row(st["b_sc"]),
                               st["W_in"], _row(st["b_in"]))
        else:
            sc, h1 = _stage_a(cur_x, s, st["W_sc"], _row(st["b_sc"]),
                              st["W_in"], _row(st["b_in"]))
        h2 = _aggr_step(h1, pos, nbr, st["W_b0"], st["b_b0"], use_sc)
        h3 = _aggr_step(h2, pos, nbr, st["W_b1"], st["b_b1"], use_sc)
        cnorm = (jnp.linalg.norm(st["p"]) + 1e-16).reshape(1, 1)
        xs, score = _stage_b(h3, sc, st["W_out"], _row(st["b_out"]),
                             st["p"].reshape(-1, 1), cnorm)
        k = int(np.ceil(ratios[i] * N))
        _, idx = lax.top_k(score[:, 0], k)
        tab = jnp.concatenate([xs, pos, score], axis=1)
        rows = _tc_gather(tab, idx)
        cur_x = rows[:, :E]
        pos = rows[:, E:E + 3]
        s = rows[:, E + 3:E + 4]
    return _head(cur_x, s, params["W_mlp"], _row(params["b_mlp"]),
                 params["W_cls"], _row(params["b_cls"]),
                 params["W_fc"], _row(params["b_fc"]))


# speedup vs baseline: 7.5118x; 1.0028x over previous
import functools

import jax
import jax.numpy as jnp
import numpy as np
from jax import lax
from jax.experimental import pallas as pl
from jax.experimental.pallas import tpu as pltpu
from jax.experimental.pallas import tpu_sc as plsc

_NEG = 0.2
_F32 = jnp.float32


def _leaky(v):
    return jnp.where(v > 0, v, _NEG * v)


def _dot(a, b):
    return jnp.dot(a, b, preferred_element_type=_F32)


def _knn16(pos8, posT):
    N = pos8.shape[0]
    Bq = 256

    def body(p_ref, qt_ref, o_ref):
        P = p_ref[...]
        p2 = jnp.sum(P * P, axis=1, keepdims=True)
        qt = qt_ref[...]
        q2 = jnp.sum(qt * qt, axis=0, keepdims=True)
        d = (q2 - 2.0 * _dot(P, qt)) + p2
        iota0 = lax.broadcasted_iota(jnp.int32, (N, Bq), 0)
        inf = jnp.float32(np.inf)
        for t in range(16):
            am = jnp.argmin(d, axis=0).astype(jnp.int32)
            o_ref[t, :] = am
            d = jnp.where(iota0 == am[None, :], inf, d)

    grid = (pl.cdiv(N, Bq),)
    return pl.pallas_call(
        body,
        grid=grid,
        in_specs=[
            pl.BlockSpec((N, 8), lambda i: (0, 0)),
            pl.BlockSpec((8, Bq), lambda i: (0, i)),
        ],
        out_specs=pl.BlockSpec((16, Bq), lambda i: (0, i)),
        out_shape=jax.ShapeDtypeStruct((16, N), jnp.int32),
    )(pos8, posT)


def _gather_rows(table, idx):
    V, D = table.shape
    B = idx.shape[0]
    NW = 32
    bpw = B // NW
    c = next(cc for cc in (512, 256, 128, 64, 32, 16, 8)
             if bpw % cc == 0 and 2 * cc * D * 4 <= 460 * 1024)
    nch = bpw // c
    npair = nch // 2
    mesh = plsc.VectorSubcoreMesh(core_axis_name="c", subcore_axis_name="s")

    @functools.partial(
        pl.kernel,
        mesh=mesh,
        out_type=jax.ShapeDtypeStruct((B, D), _F32),
        scratch_types=[
            pltpu.VMEM((bpw,), jnp.int32),
            pltpu.VMEM((c, D), _F32),
            pltpu.VMEM((c, D), _F32),
            pltpu.SemaphoreType.DMA,
            pltpu.SemaphoreType.DMA,
            pltpu.SemaphoreType.DMA,
            pltpu.SemaphoreType.DMA,
        ],
    )
    def k(tab_hbm, idx_hbm, out_hbm, idx_v, r0, r1, s0, s1, p0, p1):
        wid = lax.axis_index("s") * 2 + lax.axis_index("c")
        base = wid * bpw
        pltpu.sync_copy(idx_hbm.at[pl.ds(base, bpw)], idx_v)

        def gather(j, buf, sem):
            return pltpu.async_copy(
                tab_hbm.at[idx_v.at[pl.ds(j * c, c)]], buf, sem)

        def put(j, buf, sem):
            pltpu.async_copy(buf, out_hbm.at[pl.ds(base + j * c, c)], sem)

        def drain(buf, sem):
            pltpu.make_async_copy(tab_hbm.at[pl.ds(0, c)], buf, sem).wait()

        def pair(i2, carry):
            j0 = 2 * i2

            @pl.when(i2 > 0)
            def _():
                drain(r0, p0)
            cp0 = gather(j0, r0, s0)

            @pl.when(i2 > 0)
            def _():
                drain(r1, p1)
            cp1 = gather(j0 + 1, r1, s1)

            cp0.wait()
            put(j0, r0, p0)
            cp1.wait()
            put(j0 + 1, r1, p1)
            return carry

        lax.fori_loop(0, npair, pair, 0)
        if npair:
            drain(r0, p0)
            drain(r1, p1)
        if nch % 2:
            cp = gather(nch - 1, r0, s0)
            cp.wait()
            put(nch - 1, r0, p0)
            drain(r0, p0)

    return k(table, idx)


def _tc_gather(table, idx):
    V, D = table.shape
    K = idx.shape[0]
    B = 256

    def body(i_ref, t_ref, o_ref):
        ii = i_ref[...]
        iota = lax.broadcasted_iota(jnp.int32, (B, V), 1)
        oh = (iota == ii).astype(_F32)
        o_ref[...] = _dot(oh, t_ref[...])

    return pl.pallas_call(
        body,
        grid=(pl.cdiv(K, B),),
        in_specs=[pl.BlockSpec((B, 1), lambda i: (i, 0)),
                  pl.BlockSpec((V, D), lambda i: (0, 0))],
        out_specs=pl.BlockSpec((B, D), lambda i: (i, 0)),
        out_shape=jax.ShapeDtypeStruct((K, D), _F32),
    )(idx.reshape(-1, 1), table)


def _stage0_a(xr, Wm, bm, Wsc, bsc, Win, bin_):
    N = xr.shape[0]
    E = Wsc.shape[1]
    F1 = Win.shape[1]
    B = 512

    def body(x_ref, Wm_ref, bm_ref, Ws_ref, bs_ref, Wi_ref, bi_ref,
             sc_ref, h1_ref):
        xm = _dot(x_ref[...], Wm_ref[...]) + bm_ref[...]
        sc_ref[...] = _dot(xm, Ws_ref[...]) + bs_ref[...]
        h1_ref[...] = _leaky(_dot(xm, Wi_ref[...]) + bi_ref[...])

    full = lambda a: pl.BlockSpec(a.shape, lambda i: (0, 0))
    return pl.pallas_call(
        body,
        grid=(pl.cdiv(N, B),),
        in_specs=[pl.BlockSpec((B, xr.shape[1]), lambda i: (i, 0)),
                  full(Wm), full(bm), full(Wsc), full(bsc), full(Win),
                  full(bin_)],
        out_specs=[pl.BlockSpec((B, E), lambda i: (i, 0)),
                   pl.BlockSpec((B, F1), lambda i: (i, 0))],
        out_shape=[jax.ShapeDtypeStruct((N, E), _F32),
                   jax.ShapeDtypeStruct((N, F1), _F32)],
    )(xr, Wm, bm, Wsc, bsc, Win, bin_)


def _stage_a(x, s, Wsc, bsc, Win, bin_):
    N, I = x.shape
    E = Wsc.shape[1]
    F1 = Win.shape[1]
    B = 512

    def body(x_ref, s_ref, Ws_ref, bs_ref, Wi_ref, bi_ref, sc_ref, h1_ref):
        xm = x_ref[...] * jnp.tanh(s_ref[...])
        sc_ref[...] = _dot(xm, Ws_ref[...]) + bs_ref[...]
        h1_ref[...] = _leaky(_dot(xm, Wi_ref[...]) + bi_ref[...])

    full = lambda a: pl.BlockSpec(a.shape, lambda i: (0, 0))
    return pl.pallas_call(
        body,
        grid=(pl.cdiv(N, B),),
        in_specs=[pl.BlockSpec((B, I), lambda i: (i, 0)),
                  pl.BlockSpec((B, 1), lambda i: (i, 0)),
                  full(Wsc), full(bsc), full(Win), full(bin_)],
        out_specs=[pl.BlockSpec((B, E), lambda i: (i, 0)),
                   pl.BlockSpec((B, F1), lambda i: (i, 0))],
        out_shape=[jax.ShapeDtypeStruct((N, E), _F32),
                   jax.ShapeDtypeStruct((N, F1), _F32)],
    )(x, s, Wsc, bsc, Win, bin_)


def _aggr(tab, g, Wd, Wg, b):
    N, Dp = tab.shape
    Ds = g.shape[2]
    Fo = Wd.shape[1]
    B = 512

    def body(t_ref, g_ref, Wd_ref, Wg_ref, b_ref, o_ref):
        dpart = _dot(t_ref[...], Wd_ref[...]) + b_ref[...]
        Wgv = Wg_ref[...]
        acc = None
        for t in range(16):
            v = _leaky(_dot(g_ref[t], Wgv) + dpart)
            acc = v if acc is None else jnp.maximum(acc, v)
        o_ref[...] = acc

    full = lambda a: pl.BlockSpec(a.shape, lambda i: (0, 0))
    return pl.pallas_call(
        body,
        grid=(pl.cdiv(N, B),),
        in_specs=[pl.BlockSpec((B, Dp), lambda i: (i, 0)),
                  pl.BlockSpec((16, B, Ds), lambda i: (0, i, 0)),
                  full(Wd), full(Wg), full(b)],
        out_specs=pl.BlockSpec((B, Fo), lambda i: (i, 0)),
        out_shape=jax.ShapeDtypeStruct((N, Fo), _F32),
    )(tab, g, Wd, Wg, b)


def _stage_b(h3, sc, Wout, bout, pcol, cnorm):
    N, F3 = h3.shape
    E = Wout.shape[1]
    B = 512

    def body(h_ref, s_ref, W_ref, b_ref, p_ref, c_ref, x_ref, sco_ref):
        x = _leaky(_dot(h_ref[...], W_ref[...]) + b_ref[...] + s_ref[...])
        x_ref[...] = x
        sco_ref[...] = _dot(x, p_ref[...]) / c_ref[0, 0]

    full = lambda a: pl.BlockSpec(a.shape, lambda i: (0, 0))
    return pl.pallas_call(
        body,
        grid=(pl.cdiv(N, B),),
        in_specs=[pl.BlockSpec((B, F3), lambda i: (i, 0)),
                  pl.BlockSpec((B, E), lambda i: (i, 0)),
                  full(Wout), full(bout), full(pcol), full(cnorm)],
        out_specs=[pl.BlockSpec((B, E), lambda i: (i, 0)),
                   pl.BlockSpec((B, 1), lambda i: (i, 0))],
        out_shape=[jax.ShapeDtypeStruct((N, E), _F32),
                   jax.ShapeDtypeStruct((N, 1), _F32)],
    )(h3, sc, Wout, bout, pcol, cnorm)


def _head(x, s, Wm, bm, Wc, bc, Wf, bf):
    K = x.shape[0]

    def body(x_ref, s_ref, Wm_ref, bm_ref, Wc_ref, bc_ref, Wf_ref, bf_ref,
             o_ref):
        xm = x_ref[...] * jnp.tanh(s_ref[...])
        h = _leaky(_dot(xm, Wm_ref[...]) + bm_ref[...])
        g = jnp.max(h, axis=0, keepdims=True)
        g = _leaky(_dot(g, Wc_ref[...]) + bc_ref[...])
        o_ref[...] = _dot(g, Wf_ref[...]) + bf_ref[...]

    full = lambda a: pl.BlockSpec(a.shape, lambda i: (0, 0))
    return pl.pallas_call(
        body,
        grid=(1,),
        in_specs=[full(x), full(s), full(Wm), full(bm), full(Wc), full(bc),
                  full(Wf), full(bf)],
        out_specs=pl.BlockSpec((1, 40), lambda i: (0, 0)),
        out_shape=jax.ShapeDtypeStruct((1, 40), _F32),
    )(x, s, Wm, bm, Wc, bc, Wf, bf)


def _pad_cols(a, n):
    return jnp.pad(a, ((0, 0), (0, n - a.shape[1])))


def _pad_to_4096(idx):
    B = int(np.ceil(idx.shape[0] / 4096)) * 4096
    return jnp.pad(idx, (0, B - idx.shape[0]))


def _round128(n):
    return (n + 127) // 128 * 128


def _row(v):
    return v.reshape(1, -1)


def _split_edge_w(W, F, Dp):
    W1, W2, W3 = W[:F], W[F:2 * F], W[2 * F:]
    Wd = jnp.concatenate([W1 - W2, -W3], axis=0)
    Wg = jnp.concatenate([W2, W3], axis=0)
    pad = ((0, Dp - (F + 3)), (0, 0))
    return jnp.pad(Wd, pad), jnp.pad(Wg, pad)


def _aggr_step(h, pos, nbr, W, b, use_sc):
    N, F = h.shape
    Npad = nbr.shape[1]
    if use_sc:
        Dp = _round128(F + 3)
        tab = _pad_cols(jnp.concatenate([h, pos], axis=1), Dp)
        g = _gather_rows(tab, nbr.reshape(-1))
    else:
        Dp = (F + 3 + 7) // 8 * 8
        tab = _pad_cols(jnp.concatenate([h, pos], axis=1), Dp)
        g = _tc_gather(tab, nbr.reshape(-1))
    Wd, Wg = _split_edge_w(W, F, Dp)
    g = g.reshape(16, Npad, Dp)
    return _aggr(tab, g, Wd, Wg, _row(b))


def kernel(x, pos, batch, params):
    xr = x[:, :4, :, :].reshape(x.shape[0], -1)
    ratios = [0.5, 0.5, 0.25, 0.25]
    s = None
    cur_x = xr
    for i, st in enumerate(params["stages"]):
        N = cur_x.shape[0]
        E = st["W_sc"].shape[1]
        pos8 = _pad_cols(pos, 8)
        nbr = _knn16(pos8, pos8.T)
        use_sc = i < 2
        if use_sc:
            Npad = (N + 255) // 256 * 256
            nbr = jnp.pad(nbr, ((0, 0), (0, Npad - N)))
        if i == 0:
            sc, h1 = _stage0_a(cur_x, params["W_map"], _row(params["b_map"]),
                               st["W_sc"], _row(st["b_sc"]),
                               st["W_in"], _row(st["b_in"]))
        else:
            sc, h1 = _stage_a(cur_x, s, st["W_sc"], _row(st["b_sc"]),
                              st["W_in"], _row(st["b_in"]))
        h2 = _aggr_step(h1, pos, nbr, st["W_b0"], st["b_b0"], use_sc)
        h3 = _aggr_step(h2, pos, nbr, st["W_b1"], st["b_b1"], use_sc)
        cnorm = (jnp.linalg.norm(st["p"]) + 1e-16).reshape(1, 1)
        xs, score = _stage_b(h3, sc, st["W_out"], _row(st["b_out"]),
                             st["p"].reshape(-1, 1), cnorm)
        k = int(np.ceil(ratios[i] * N))
        _, idx = lax.top_k(score[:, 0], k)
        tab = jnp.concatenate([xs, pos, score], axis=1)
        rows = _tc_gather(tab, idx)
        cur_x = rows[:, :E]
        pos = rows[:, E:E + 3]
        s = rows[:, E + 3:E + 4]
    return _head(cur_x, s, params["W_mlp"], _row(params["b_mlp"]),
                 params["W_cls"], _row(params["b_cls"]),
                 params["W_fc"], _row(params["b_fc"]))
